# baseline (speedup 1.0000x reference)
import functools

import jax
import jax.numpy as jnp
from jax import lax
from jax.experimental import pallas as pl
from jax.experimental.pallas import tpu as pltpu
from jax.experimental.pallas import tpu_sc as plsc

L_SEQ, B, NFEAT = 200, 1024, 3
N = L_SEQ * B
NW = 32
BPW = N // NW
GR = 64
NGROUPS = BPW // GR
NB = 4

_mesh = plsc.VectorSubcoreMesh(core_axis_name="c", subcore_axis_name="s")


@functools.partial(
    pl.kernel,
    out_type=jax.ShapeDtypeStruct((N, 256), jnp.float32),
    mesh=_mesh,
    compiler_params=pltpu.CompilerParams(disable_bounds_checks=True),
    scratch_types=[
        pltpu.VMEM((NGROUPS // 2, 2 * GR), jnp.int32),
        pltpu.VMEM((NGROUPS // 2, 2 * GR), jnp.int32),
        pltpu.VMEM((NGROUPS // 2, 2 * GR), jnp.int32),
        pltpu.VMEM((NB, GR, 256), jnp.float32),
        pltpu.VMEM((NB, GR, 128), jnp.float32),
        [pltpu.SemaphoreType.DMA] * NB,
        [pltpu.SemaphoreType.DMA] * NB,
        [pltpu.SemaphoreType.DMA] * NB,
    ],
)
def _embed_sc(w0, w12, i0, i1, i2, out,
              i0_v, i1_v, i2_v, obuf, cbuf, gsems, csems, ssems):
    wid = lax.axis_index("s") * 2 + lax.axis_index("c")
    base = wid * BPW

    pltpu.sync_copy(i0.at[wid], i0_v)
    pltpu.sync_copy(i1.at[wid], i1_v)
    pltpu.sync_copy(i2.at[wid], i2_v)

    def _irow(iv, g):
        return iv.at[g >> 1, pl.ds((g & 1) * GR, GR)]

    def _fire_gathers(g, slot):
        pltpu.async_copy(w0.at[_irow(i0_v, g)], obuf.at[slot, :, pl.ds(0, 128)],
                         gsems[slot])
        pltpu.async_copy(w12.at[_irow(i1_v, g)],
                         obuf.at[slot, :, pl.ds(128, 128)], gsems[slot])
        pltpu.async_copy(w12.at[_irow(i2_v, g)], cbuf.at[slot], csems[slot])

    def _wait_gathers(g, slot):
        pltpu.make_async_copy(w0.at[_irow(i0_v, g)],
                              obuf.at[slot, :, pl.ds(0, 128)],
                              gsems[slot]).wait()
        pltpu.make_async_copy(w12.at[_irow(i1_v, g)],
                              obuf.at[slot, :, pl.ds(128, 128)],
                              gsems[slot]).wait()
        pltpu.make_async_copy(w12.at[_irow(i2_v, g)], cbuf.at[slot],
                              csems[slot]).wait()

    def _merge(slot):
        @pl.loop(0, GR, unroll=4)
        def _rows(r):
            for q in range(4):
                obuf[slot, r, pl.ds(192 + q * 16, 16)] = \
                    cbuf[slot, r, pl.ds(64 + q * 16, 16)]

    def _fire_scatter(g, slot):
        pltpu.async_copy(obuf.at[slot], out.at[pl.ds(base + g * GR, GR)],
                         ssems[slot])

    def _wait_scatter(g, slot):
        pltpu.make_async_copy(obuf.at[slot],
                              out.at[pl.ds(base + g * GR, GR)],
                              ssems[slot]).wait()

    for g in range(NB - 1):
        _fire_gathers(g, g)

    @pl.loop(0, NGROUPS, step=NB)
    def _groups(g0):
        for slot in range(NB):
            g = g0 + slot
            _wait_gathers(g, slot)
            _merge(slot)
            _fire_scatter(g, slot)
            slot_prev = (slot - 1) % NB
            @pl.when(g + NB - 1 < NGROUPS)
            def _():
                @pl.when(g >= 1)
                def _():
                    _wait_scatter(g - 1, slot_prev)
                _fire_gathers(g + NB - 1, slot_prev)

    for g in range(NGROUPS - NB, NGROUPS):
        _wait_scatter(g, g % NB)


def kernel(input, W0, W1, W2):
    inp = input.reshape(N, NFEAT).astype(jnp.int32)
    i0 = inp[:, 0].reshape(NW, NGROUPS // 2, 2 * GR)
    i1 = inp[:, 1].reshape(NW, NGROUPS // 2, 2 * GR)
    i2 = inp[:, 2].reshape(NW, NGROUPS // 2, 2 * GR)
    w12 = jnp.concatenate([W1, W2], axis=1)
    out = _embed_sc(W0, w12, i0, i1, i2)
    return out.reshape(L_SEQ, B, 256)
